# baseline (speedup 1.0000x reference)
import functools

import jax
import jax.numpy as jnp
from jax import lax
from jax.experimental import pallas as pl
from jax.experimental.pallas import tpu as pltpu
from jax.experimental.pallas import tpu_sc as plsc

NUM_CORES = 2
NUM_SUBCORES = 16
NUM_WORKERS = NUM_CORES * NUM_SUBCORES
CHUNK = 128
NBUF = 8
PREFETCH = 6
WT_BLOCK = 8192


def _transpose_table(weight_t, V, D):
    def body(in_ref, out_ref):
        out_ref[...] = in_ref[...].T

    return pl.pallas_call(
        body,
        grid=(pl.cdiv(V, WT_BLOCK),),
        in_specs=[pl.BlockSpec((D, WT_BLOCK), lambda i: (0, i))],
        out_specs=pl.BlockSpec((WT_BLOCK, D), lambda i: (i, 0)),
        out_shape=jax.ShapeDtypeStruct((V, D), jnp.float32),
    )(weight_t)


def _transpose_out(flat, S, B, D):
    x = flat.reshape(S, B, D)

    def body(in_ref, out_ref):
        out_ref[0] = in_ref[0].T

    return pl.pallas_call(
        body,
        grid=(S,),
        in_specs=[pl.BlockSpec((1, B, D), lambda s: (s, 0, 0))],
        out_specs=pl.BlockSpec((1, D, B), lambda s: (s, 0, 0)),
        out_shape=jax.ShapeDtypeStruct((S, D, B), jnp.float32),
    )(x)


def kernel(token_ids, weight):
    B, S = token_ids.shape
    V, D = weight.shape
    total = B * S
    per_w = total // NUM_WORKERS
    n_chunks = per_w // CHUNK
    n_groups = n_chunks // NBUF
    idx = token_ids.T.reshape(NUM_WORKERS, n_chunks, CHUNK).astype(jnp.int32)
    table = _transpose_table(weight.T, V, D)

    mesh = plsc.VectorSubcoreMesh(core_axis_name="c", subcore_axis_name="s")

    @functools.partial(
        pl.kernel,
        mesh=mesh,
        out_type=jax.ShapeDtypeStruct((total, D), jnp.float32),
        scratch_types=[
            pltpu.VMEM((n_chunks, CHUNK), jnp.int32),
            pltpu.VMEM((NBUF, CHUNK, D), jnp.float32),
            [pltpu.SemaphoreType.DMA] * NBUF,
            [pltpu.SemaphoreType.DMA] * NBUF,
        ],
        compiler_params=pltpu.CompilerParams(use_tc_tiling_on_sc=False),
    )
    def emb(table_hbm, idx_hbm, out_hbm, idx_v, rows_v, gsems, wsems):
        wid = lax.axis_index("s") * NUM_CORES + lax.axis_index("c")
        base = wid * per_w
        pltpu.sync_copy(idx_hbm.at[wid], idx_v)

        def gather(j, b, sem):
            return pltpu.make_async_copy(
                table_hbm.at[idx_v.at[j]], rows_v.at[b], sem
            )

        def writeback(j, b, sem):
            return pltpu.make_async_copy(
                rows_v.at[b], out_hbm.at[pl.ds(base + j * CHUNK, CHUNK)], sem
            )

        for b in range(PREFETCH):
            gather(b, b, gsems[b]).start()

        def body(g, carry):
            for b in range(NBUF):
                j = g * NBUF + b
                bp = (b + PREFETCH) % NBUF
                @pl.when(j + PREFETCH - NBUF >= 0)
                def _():
                    writeback(j + PREFETCH - NBUF, bp, wsems[bp]).wait()

                @pl.when(j + PREFETCH < n_chunks)
                def _():
                    gather(j + PREFETCH, bp, gsems[bp]).start()

                gather(j, b, gsems[b]).wait()
                writeback(j, b, wsems[b]).start()
            return carry

        lax.fori_loop(0, n_groups, body, 0)

        for j in range(n_chunks - (NBUF - PREFETCH), n_chunks):
            b = j % NBUF
            writeback(j, b, wsems[b]).wait()

    out_flat = emb(table, idx)
    out_t = _transpose_out(out_flat, S, B, D)
    return out_t.transpose(2, 0, 1)


# speedup vs baseline: 1.5974x; 1.5974x over previous
import functools

import jax
import jax.numpy as jnp
from jax import lax
from jax.experimental import pallas as pl
from jax.experimental.pallas import tpu as pltpu
from jax.experimental.pallas import tpu_sc as plsc

NUM_CORES = 2
NUM_SUBCORES = 16
NUM_WORKERS = NUM_CORES * NUM_SUBCORES
CHUNK = 128
NBUF = 8
PREFETCH = 6
WT_BLOCK = 8192
BW = 128


def _format_out(flat128, S, B, D):
    row_bytes = S * D
    rows_per_w = BW * row_bytes // 128
    k = row_bytes // 128

    def body(in_ref, out_ref):
        x = in_ref[...].reshape(BW, k, 128)
        y = jnp.transpose(x, (1, 2, 0))
        out_ref[...] = y.reshape(row_bytes, BW)

    return pl.pallas_call(
        body,
        grid=(B // BW,),
        in_specs=[pl.BlockSpec((rows_per_w, 128), lambda i: (i, 0))],
        out_specs=pl.BlockSpec((row_bytes, BW), lambda i: (0, i)),
        out_shape=jax.ShapeDtypeStruct((row_bytes, B), jnp.float32),
    )(flat128)


def kernel(token_ids, weight):
    B, S = token_ids.shape
    V, D = weight.shape
    total = B * S
    per_w = total // NUM_WORKERS
    n_chunks = per_w // CHUNK
    n_groups = n_chunks // NBUF
    idx = token_ids.reshape(NUM_WORKERS, n_chunks, CHUNK).astype(jnp.int32)

    mesh = plsc.VectorSubcoreMesh(core_axis_name="c", subcore_axis_name="s")

    @functools.partial(
        pl.kernel,
        mesh=mesh,
        out_type=jax.ShapeDtypeStruct((total, D), jnp.float32),
        scratch_types=[
            pltpu.VMEM((n_chunks, CHUNK), jnp.int32),
            pltpu.VMEM((NBUF, CHUNK, D), jnp.float32),
            [pltpu.SemaphoreType.DMA] * NBUF,
            [pltpu.SemaphoreType.DMA] * NBUF,
        ],
        compiler_params=pltpu.CompilerParams(use_tc_tiling_on_sc=False),
    )
    def emb(table_hbm, idx_hbm, out_hbm, idx_v, rows_v, gsems, wsems):
        wid = lax.axis_index("s") * NUM_CORES + lax.axis_index("c")
        base = wid * per_w
        pltpu.sync_copy(idx_hbm.at[wid], idx_v)

        def gather(j, b, sem):
            return pltpu.make_async_copy(
                table_hbm.at[idx_v.at[j]], rows_v.at[b], sem
            )

        def writeback(j, b, sem):
            return pltpu.make_async_copy(
                rows_v.at[b], out_hbm.at[pl.ds(base + j * CHUNK, CHUNK)], sem
            )

        for b in range(PREFETCH):
            gather(b, b, gsems[b]).start()

        def body(g, carry):
            for b in range(NBUF):
                j = g * NBUF + b
                bp = (b + PREFETCH) % NBUF
                @pl.when(j + PREFETCH - NBUF >= 0)
                def _():
                    writeback(j + PREFETCH - NBUF, bp, wsems[bp]).wait()

                @pl.when(j + PREFETCH < n_chunks)
                def _():
                    gather(j + PREFETCH, bp, gsems[bp]).start()

                gather(j, b, gsems[b]).wait()
                writeback(j, b, wsems[b]).start()
            return carry

        lax.fori_loop(0, n_groups, body, 0)

        for j in range(n_chunks - (NBUF - PREFETCH), n_chunks):
            b = j % NBUF
            writeback(j, b, wsems[b]).wait()

    out_flat = emb(weight, idx)
    out_t = _format_out(out_flat.reshape(total * D // 128, 128), S, B, D)
    return out_t.reshape(S, D, B).transpose(2, 0, 1)


# speedup vs baseline: 1.7764x; 1.1121x over previous
import functools

import jax
import jax.numpy as jnp
from jax import lax
from jax.experimental import pallas as pl
from jax.experimental.pallas import tpu as pltpu
from jax.experimental.pallas import tpu_sc as plsc

NUM_CORES = 2
NUM_SUBCORES = 16
NUM_WORKERS = NUM_CORES * NUM_SUBCORES
CHUNK = 128
NBUF = 8
PREFETCH = 6
BW = 256


def _format_out(flat128, S, B, D):
    row_bytes = S * D
    rows_per_w = BW * row_bytes // 128
    k = row_bytes // 128

    def body(in_ref, out_ref):
        x = in_ref[...].reshape(BW, k, 128)
        for kk in range(k):
            out_ref[pl.ds(kk * 128, 128), :] = x[:, kk, :].T

    return pl.pallas_call(
        body,
        grid=(B // BW,),
        in_specs=[pl.BlockSpec((rows_per_w, 128), lambda i: (i, 0))],
        out_specs=pl.BlockSpec((row_bytes, BW), lambda i: (0, i)),
        out_shape=jax.ShapeDtypeStruct((row_bytes, B), jnp.float32),
    )(flat128)


def kernel(token_ids, weight):
    B, S = token_ids.shape
    V, D = weight.shape
    total = B * S
    per_w = total // NUM_WORKERS
    n_chunks = per_w // CHUNK
    n_groups = n_chunks // NBUF
    idx = token_ids.reshape(NUM_WORKERS, n_chunks, CHUNK).astype(jnp.int32)

    mesh = plsc.VectorSubcoreMesh(core_axis_name="c", subcore_axis_name="s")

    @functools.partial(
        pl.kernel,
        mesh=mesh,
        out_type=jax.ShapeDtypeStruct((total, D), jnp.float32),
        scratch_types=[
            pltpu.VMEM((n_chunks, CHUNK), jnp.int32),
            pltpu.VMEM((NBUF, CHUNK, D), jnp.float32),
            [pltpu.SemaphoreType.DMA] * NBUF,
            [pltpu.SemaphoreType.DMA] * NBUF,
        ],
        compiler_params=pltpu.CompilerParams(use_tc_tiling_on_sc=False),
    )
    def emb(table_hbm, idx_hbm, out_hbm, idx_v, rows_v, gsems, wsems):
        wid = lax.axis_index("s") * NUM_CORES + lax.axis_index("c")
        base = wid * per_w
        pltpu.sync_copy(idx_hbm.at[wid], idx_v)

        def gather(j, b, sem):
            return pltpu.make_async_copy(
                table_hbm.at[idx_v.at[j]], rows_v.at[b], sem
            )

        def writeback(j, b, sem):
            return pltpu.make_async_copy(
                rows_v.at[b], out_hbm.at[pl.ds(base + j * CHUNK, CHUNK)], sem
            )

        for b in range(PREFETCH):
            gather(b, b, gsems[b]).start()

        def body(g, carry):
            for b in range(NBUF):
                j = g * NBUF + b
                bp = (b + PREFETCH) % NBUF
                @pl.when(j + PREFETCH - NBUF >= 0)
                def _():
                    writeback(j + PREFETCH - NBUF, bp, wsems[bp]).wait()

                @pl.when(j + PREFETCH < n_chunks)
                def _():
                    gather(j + PREFETCH, bp, gsems[bp]).start()

                gather(j, b, gsems[b]).wait()
                writeback(j, b, wsems[b]).start()
            return carry

        lax.fori_loop(0, n_groups, body, 0)

        for j in range(n_chunks - (NBUF - PREFETCH), n_chunks):
            b = j % NBUF
            writeback(j, b, wsems[b]).wait()

    out_flat = emb(weight, idx)
    out_t = _format_out(out_flat.reshape(total * D // 128, 128), S, B, D)
    return out_t.reshape(S, D, B).transpose(2, 0, 1)
